# Initial kernel scaffold; baseline (speedup 1.0000x reference)
import jax
import jax.numpy as jnp
from jax.experimental import pallas as pl


def kernel(score_map, delta_map, anchors):
    raise NotImplementedError("write your pallas kernel here")


# speedup vs baseline: 11.6985x; 11.6985x over previous
import functools
import jax
import jax.numpy as jnp
from jax.experimental import pallas as pl
from jax.experimental.pallas import tpu as pltpu

MAX_OUT = 300
IOU_T = 0.7
SCORE_T = 0.0
PROP_T = 0.5
LANES = 128


def _nms_body(score_ref, delta_ref, anch_ref, out_ref,
              sc_ref, bx0_ref, bx1_ref, by0_ref, by1_ref,
              kx0_ref, kx1_ref, ky0_ref, ky1_ref):
    R = sc_ref.shape[0]
    N = R * LANES

    a0 = anch_ref[0]
    a1 = anch_ref[1]
    a2 = anch_ref[2]
    a3 = anch_ref[3]
    xa = (a0 + a1) * 0.5
    ya = (a2 + a3) * 0.5
    wa = a1 - a0
    ha = a3 - a2
    tx = delta_ref[0, 0]
    ty = delta_ref[0, 1]
    tw = delta_ref[0, 2]
    th = delta_ref[0, 3]
    x = tx * wa + xa
    y = ty * ha + ya
    w = jnp.exp(tw) * wa
    h = jnp.exp(th) * ha
    bx0_ref[...] = jnp.clip(x - w * 0.5, 0.0, 1.0)
    bx1_ref[...] = jnp.clip(x + w * 0.5, 0.0, 1.0)
    by0_ref[...] = jnp.clip(y - h * 0.5, 0.0, 1.0)
    by1_ref[...] = jnp.clip(y + h * 0.5, 0.0, 1.0)

    s = score_ref[0]
    sc_ref[...] = jnp.where(s > PROP_T, s, -1.0)

    kx0_ref[...] = jnp.zeros_like(kx0_ref)
    kx1_ref[...] = jnp.zeros_like(kx1_ref)
    ky0_ref[...] = jnp.zeros_like(ky0_ref)
    ky1_ref[...] = jnp.zeros_like(ky1_ref)
    out_ref[0] = jnp.zeros_like(out_ref[0])

    flat = (jax.lax.broadcasted_iota(jnp.int32, (R, LANES), 0) * LANES
            + jax.lax.broadcasted_iota(jnp.int32, (R, LANES), 1))
    lane = jax.lax.broadcasted_iota(jnp.int32, (1, LANES), 1)

    def cond(carry):
        k, done = carry
        return jnp.logical_and(k < MAX_OUT, jnp.logical_not(done))

    def body(carry):
        k, _ = carry
        s_all = sc_ref[...]
        m = jnp.max(s_all)
        j = jnp.min(jnp.where(s_all == m, flat, N))
        r = j // LANES
        c = j % LANES

        srow = sc_ref[pl.ds(r, 1), :]
        sc_ref[pl.ds(r, 1), :] = jnp.where(lane == c, -1.0, srow)

        onehot = (lane == c)
        x0 = jnp.sum(jnp.where(onehot, bx0_ref[pl.ds(r, 1), :], 0.0))
        x1 = jnp.sum(jnp.where(onehot, bx1_ref[pl.ds(r, 1), :], 0.0))
        y0 = jnp.sum(jnp.where(onehot, by0_ref[pl.ds(r, 1), :], 0.0))
        y1 = jnp.sum(jnp.where(onehot, by1_ref[pl.ds(r, 1), :], 0.0))

        kx0 = kx0_ref[...]
        kx1 = kx1_ref[...]
        ky0 = ky0_ref[...]
        ky1 = ky1_ref[...]
        iw = jnp.maximum(jnp.minimum(x1, kx1) - jnp.maximum(x0, kx0), 0.0)
        ih = jnp.maximum(jnp.minimum(y1, ky1) - jnp.maximum(y0, ky0), 0.0)
        inter = iw * ih
        area = (x1 - x0) * (y1 - y0)
        areas = (kx1 - kx0) * (ky1 - ky0)
        iou = inter / (area + areas - inter + 1e-9)
        overlap = jnp.any(iou > IOU_T)

        alive = m > SCORE_T
        keep = jnp.logical_and(alive, jnp.logical_not(overlap))

        @pl.when(keep)
        def _():
            kr = k // LANES
            kc = k % LANES
            ksel = (lane == kc)
            kx0_ref[pl.ds(kr, 1), :] = jnp.where(ksel, x0, kx0_ref[pl.ds(kr, 1), :])
            kx1_ref[pl.ds(kr, 1), :] = jnp.where(ksel, x1, kx1_ref[pl.ds(kr, 1), :])
            ky0_ref[pl.ds(kr, 1), :] = jnp.where(ksel, y0, ky0_ref[pl.ds(kr, 1), :])
            ky1_ref[pl.ds(kr, 1), :] = jnp.where(ksel, y1, ky1_ref[pl.ds(kr, 1), :])
            row = jnp.where(lane == 0, x0,
                  jnp.where(lane == 1, x1,
                  jnp.where(lane == 2, y0,
                  jnp.where(lane == 3, y1, 0.0))))
            out_ref[0, pl.ds(k, 1), :] = row

        return k + keep.astype(jnp.int32), jnp.logical_not(alive)

    jax.lax.while_loop(cond, body, (jnp.int32(0), jnp.bool_(False)))


@functools.partial(jax.jit, static_argnames=("interpret",))
def kernel(score_map, delta_map, anchors, interpret=False):
    B, H, W, A = score_map.shape
    N = H * W * A
    R = N // LANES
    assert N % LANES == 0

    scores = score_map.reshape(B, R, LANES)
    deltas = delta_map.reshape(B, N, 4).transpose(0, 2, 1).reshape(B, 4, R, LANES)
    anch = anchors.reshape(N, 4).T.reshape(4, R, LANES)

    out = pl.pallas_call(
        _nms_body,
        grid=(B,),
        in_specs=[
            pl.BlockSpec((1, R, LANES), lambda b: (b, 0, 0)),
            pl.BlockSpec((1, 4, R, LANES), lambda b: (b, 0, 0, 0)),
            pl.BlockSpec((4, R, LANES), lambda b: (0, 0, 0)),
        ],
        out_specs=pl.BlockSpec((1, MAX_OUT, LANES), lambda b: (b, 0, 0)),
        out_shape=jax.ShapeDtypeStruct((B, MAX_OUT, LANES), jnp.float32),
        scratch_shapes=[
            pltpu.VMEM((R, LANES), jnp.float32),
            pltpu.VMEM((R, LANES), jnp.float32),
            pltpu.VMEM((R, LANES), jnp.float32),
            pltpu.VMEM((R, LANES), jnp.float32),
            pltpu.VMEM((R, LANES), jnp.float32),
            pltpu.VMEM((8, LANES), jnp.float32),
            pltpu.VMEM((8, LANES), jnp.float32),
            pltpu.VMEM((8, LANES), jnp.float32),
            pltpu.VMEM((8, LANES), jnp.float32),
        ],
        interpret=interpret,
    )(scores, deltas, anch)
    return out[:, :, :4]


# speedup vs baseline: 11.7993x; 1.0086x over previous
import functools
import jax
import jax.numpy as jnp
from jax.experimental import pallas as pl
from jax.experimental.pallas import tpu as pltpu

MAX_OUT = 300
IOU_T = 0.7
SCORE_T = 0.0
PROP_T = 0.5
LANES = 128


def _nms_body(score_ref, delta_ref, anch_ref, out_ref,
              sc_ref, bx0_ref, bx1_ref, by0_ref, by1_ref,
              kx0_ref, kx1_ref, ky0_ref, ky1_ref):
    B, R, _ = sc_ref.shape
    N = R * LANES

    for i in range(B):
        a0 = anch_ref[0]
        a1 = anch_ref[1]
        a2 = anch_ref[2]
        a3 = anch_ref[3]
        xa = (a0 + a1) * 0.5
        ya = (a2 + a3) * 0.5
        wa = a1 - a0
        ha = a3 - a2
        tx = delta_ref[i, 0]
        ty = delta_ref[i, 1]
        tw = delta_ref[i, 2]
        th = delta_ref[i, 3]
        x = tx * wa + xa
        y = ty * ha + ya
        w = jnp.exp(tw) * wa
        h = jnp.exp(th) * ha
        bx0_ref[i] = jnp.clip(x - w * 0.5, 0.0, 1.0)
        bx1_ref[i] = jnp.clip(x + w * 0.5, 0.0, 1.0)
        by0_ref[i] = jnp.clip(y - h * 0.5, 0.0, 1.0)
        by1_ref[i] = jnp.clip(y + h * 0.5, 0.0, 1.0)
        s = score_ref[i]
        sc_ref[i] = jnp.where(s > PROP_T, s, -1.0)

    kx0_ref[...] = jnp.zeros_like(kx0_ref)
    kx1_ref[...] = jnp.zeros_like(kx1_ref)
    ky0_ref[...] = jnp.zeros_like(ky0_ref)
    ky1_ref[...] = jnp.zeros_like(ky1_ref)
    out_ref[...] = jnp.zeros_like(out_ref)

    flat = (jax.lax.broadcasted_iota(jnp.int32, (R, LANES), 0) * LANES
            + jax.lax.broadcasted_iota(jnp.int32, (R, LANES), 1))
    lane = jax.lax.broadcasted_iota(jnp.int32, (1, LANES), 1)

    def argmax(i):
        s_all = sc_ref[i]
        m = jnp.max(s_all)
        j = jnp.min(jnp.where(s_all == m, flat, N))
        return m, j

    m0, j0 = argmax(0)
    m1, j1 = argmax(1)

    def cond(carry):
        k0, m0, _, k1, m1, _ = carry
        a0 = jnp.logical_and(k0 < MAX_OUT, m0 > SCORE_T)
        a1 = jnp.logical_and(k1 < MAX_OUT, m1 > SCORE_T)
        return jnp.logical_or(a0, a1)

    def step(i, k, m, j):
        active = jnp.logical_and(k < MAX_OUT, m > SCORE_T)
        r = j // LANES
        c = j % LANES
        onehot = (lane == c)

        @pl.when(active)
        def _():
            srow = sc_ref[i, pl.ds(r, 1), :]
            sc_ref[i, pl.ds(r, 1), :] = jnp.where(onehot, -1.0, srow)

        x0 = jnp.sum(jnp.where(onehot, bx0_ref[i, pl.ds(r, 1), :], 0.0))
        x1 = jnp.sum(jnp.where(onehot, bx1_ref[i, pl.ds(r, 1), :], 0.0))
        y0 = jnp.sum(jnp.where(onehot, by0_ref[i, pl.ds(r, 1), :], 0.0))
        y1 = jnp.sum(jnp.where(onehot, by1_ref[i, pl.ds(r, 1), :], 0.0))
        kx0 = kx0_ref[i]
        kx1 = kx1_ref[i]
        ky0 = ky0_ref[i]
        ky1 = ky1_ref[i]
        iw = jnp.maximum(jnp.minimum(x1, kx1) - jnp.maximum(x0, kx0), 0.0)
        ih = jnp.maximum(jnp.minimum(y1, ky1) - jnp.maximum(y0, ky0), 0.0)
        inter = iw * ih
        area = (x1 - x0) * (y1 - y0)
        areas = (kx1 - kx0) * (ky1 - ky0)
        iou = inter / (area + areas - inter + 1e-9)
        overlap = jnp.any(iou > IOU_T)
        keep = jnp.logical_and(active, jnp.logical_not(overlap))

        @pl.when(keep)
        def _():
            kr = k // LANES
            kc = k % LANES
            ksel = (lane == kc)
            kx0_ref[i, pl.ds(kr, 1), :] = jnp.where(ksel, x0, kx0_ref[i, pl.ds(kr, 1), :])
            kx1_ref[i, pl.ds(kr, 1), :] = jnp.where(ksel, x1, kx1_ref[i, pl.ds(kr, 1), :])
            ky0_ref[i, pl.ds(kr, 1), :] = jnp.where(ksel, y0, ky0_ref[i, pl.ds(kr, 1), :])
            ky1_ref[i, pl.ds(kr, 1), :] = jnp.where(ksel, y1, ky1_ref[i, pl.ds(kr, 1), :])
            row = jnp.where(lane == 0, x0,
                  jnp.where(lane == 1, x1,
                  jnp.where(lane == 2, y0,
                  jnp.where(lane == 3, y1, 0.0))))
            out_ref[i, pl.ds(k, 1), :] = row

        m2, j2 = argmax(i)
        return k + keep.astype(jnp.int32), m2, j2

    def body(carry):
        k0, m0, j0, k1, m1, j1 = carry
        k0, m0, j0 = step(0, k0, m0, j0)
        k1, m1, j1 = step(1, k1, m1, j1)
        return k0, m0, j0, k1, m1, j1

    jax.lax.while_loop(cond, body,
                       (jnp.int32(0), m0, j0, jnp.int32(0), m1, j1))


@functools.partial(jax.jit, static_argnames=("interpret",))
def kernel(score_map, delta_map, anchors, interpret=False):
    B, H, W, A = score_map.shape
    N = H * W * A
    R = N // LANES
    assert N % LANES == 0

    scores = score_map.reshape(B, R, LANES)
    deltas = delta_map.reshape(B, N, 4).transpose(0, 2, 1).reshape(B, 4, R, LANES)
    anch = anchors.reshape(N, 4).T.reshape(4, R, LANES)

    out = pl.pallas_call(
        _nms_body,
        out_shape=jax.ShapeDtypeStruct((B, MAX_OUT, LANES), jnp.float32),
        scratch_shapes=[
            pltpu.VMEM((B, R, LANES), jnp.float32),
            pltpu.VMEM((B, R, LANES), jnp.float32),
            pltpu.VMEM((B, R, LANES), jnp.float32),
            pltpu.VMEM((B, R, LANES), jnp.float32),
            pltpu.VMEM((B, R, LANES), jnp.float32),
            pltpu.VMEM((B, 8, LANES), jnp.float32),
            pltpu.VMEM((B, 8, LANES), jnp.float32),
            pltpu.VMEM((B, 8, LANES), jnp.float32),
            pltpu.VMEM((B, 8, LANES), jnp.float32),
        ],
        interpret=interpret,
    )(scores, deltas, anch)
    return out[:, :, :4]


# speedup vs baseline: 36.0762x; 3.0575x over previous
import functools
import jax
import jax.numpy as jnp
from jax.experimental import pallas as pl
from jax.experimental.pallas import tpu as pltpu

MAX_OUT = 300
IOU_T = 0.7
SCORE_T = 0.0
PROP_T = 0.5
LANES = 128
UNROLL = 4


def _allmax(v):
    return jnp.max(v, axis=1, keepdims=True)


def _allmin(v):
    return jnp.min(v, axis=1, keepdims=True)


def _nms_body(score_ref, delta_ref, anch_ref,
              kx0_ref, kx1_ref, ky0_ref, ky1_ref,
              sc_ref, bx0_ref, bx1_ref, by0_ref, by1_ref):
    B, R, _ = sc_ref.shape
    N = R * LANES

    for i in range(B):
        a0 = anch_ref[0]
        a1 = anch_ref[1]
        a2 = anch_ref[2]
        a3 = anch_ref[3]
        xa = (a0 + a1) * 0.5
        ya = (a2 + a3) * 0.5
        wa = a1 - a0
        ha = a3 - a2
        tx = delta_ref[i, 0]
        ty = delta_ref[i, 1]
        tw = delta_ref[i, 2]
        th = delta_ref[i, 3]
        x = tx * wa + xa
        y = ty * ha + ya
        w = jnp.exp(tw) * wa
        h = jnp.exp(th) * ha
        bx0_ref[i] = jnp.clip(x - w * 0.5, 0.0, 1.0)
        bx1_ref[i] = jnp.clip(x + w * 0.5, 0.0, 1.0)
        by0_ref[i] = jnp.clip(y - h * 0.5, 0.0, 1.0)
        by1_ref[i] = jnp.clip(y + h * 0.5, 0.0, 1.0)
        s = score_ref[i]
        sc_ref[i] = jnp.where(s > PROP_T, s, -1.0)

    kx0_ref[...] = jnp.zeros_like(kx0_ref)
    kx1_ref[...] = jnp.zeros_like(kx1_ref)
    ky0_ref[...] = jnp.zeros_like(ky0_ref)
    ky1_ref[...] = jnp.zeros_like(ky1_ref)

    flat = (jax.lax.broadcasted_iota(jnp.int32, (R, LANES), 0) * LANES
            + jax.lax.broadcasted_iota(jnp.int32, (R, LANES), 1))
    flat8 = (jax.lax.broadcasted_iota(jnp.int32, (8, LANES), 0) * LANES
             + jax.lax.broadcasted_iota(jnp.int32, (8, LANES), 1))

    def next_cand(sc_vals):
        m = _allmax(jnp.max(sc_vals, axis=0, keepdims=True))
        cand = jnp.where(sc_vals == m, flat, N)
        j = _allmin(jnp.min(cand, axis=0, keepdims=True))
        return m, j

    def extract(arr, jmask):
        return _allmax(jnp.max(jnp.where(jmask, arr, -1.0), axis=0,
                               keepdims=True))

    def step(i, k, m, j):
        sc_vals = sc_ref[i]
        jmask = flat == j
        sc_new = jnp.where(jmask, -1.0, sc_vals)
        sc_ref[i] = sc_new

        x0 = extract(bx0_ref[i], jmask)
        x1 = extract(bx1_ref[i], jmask)
        y0 = extract(by0_ref[i], jmask)
        y1 = extract(by1_ref[i], jmask)
        kx0 = kx0_ref[i]
        kx1 = kx1_ref[i]
        ky0 = ky0_ref[i]
        ky1 = ky1_ref[i]
        iw = jnp.maximum(jnp.minimum(x1, kx1) - jnp.maximum(x0, kx0), 0.0)
        ih = jnp.maximum(jnp.minimum(y1, ky1) - jnp.maximum(y0, ky0), 0.0)
        inter = iw * ih
        area = (x1 - x0) * (y1 - y0)
        areas = (kx1 - kx0) * (ky1 - ky0)
        iou = inter / (area + areas - inter + 1e-9)
        ov = _allmax(jnp.max(jnp.where(iou > IOU_T, 1.0, 0.0), axis=0,
                             keepdims=True))
        active = jnp.logical_and(k < MAX_OUT, m > SCORE_T)
        keep = jnp.logical_and(active, ov < 0.5)

        sel = jnp.logical_and(flat8 == k, keep)
        kx0_ref[i] = jnp.where(sel, x0, kx0)
        kx1_ref[i] = jnp.where(sel, x1, kx1)
        ky0_ref[i] = jnp.where(sel, y0, ky0)
        ky1_ref[i] = jnp.where(sel, y1, ky1)

        m2, j2 = next_cand(sc_new)
        return k + keep.astype(jnp.int32), m2, j2

    def cond(carry):
        k0, m0, _, k1, m1, _ = carry
        a0 = jnp.logical_and(k0 < MAX_OUT, m0 > SCORE_T)
        a1 = jnp.logical_and(k1 < MAX_OUT, m1 > SCORE_T)
        return jnp.any(jnp.logical_or(a0, a1))

    def body(carry):
        k0, m0, j0, k1, m1, j1 = carry
        for _ in range(UNROLL):
            k0, m0, j0 = step(0, k0, m0, j0)
            k1, m1, j1 = step(1, k1, m1, j1)
        return k0, m0, j0, k1, m1, j1

    m0, j0 = next_cand(sc_ref[0])
    m1, j1 = next_cand(sc_ref[1])
    zk = jnp.zeros((1, 1), jnp.int32)
    jax.lax.while_loop(cond, body, (zk, m0, j0, zk, m1, j1))


@functools.partial(jax.jit, static_argnames=("interpret",))
def kernel(score_map, delta_map, anchors, interpret=False):
    B, H, W, A = score_map.shape
    N = H * W * A
    R = N // LANES
    assert N % LANES == 0

    scores = score_map.reshape(B, R, LANES)
    deltas = delta_map.reshape(B, N, 4).transpose(0, 2, 1).reshape(B, 4, R, LANES)
    anch = anchors.reshape(N, 4).T.reshape(4, R, LANES)

    shp = jax.ShapeDtypeStruct((B, 8, LANES), jnp.float32)
    kx0, kx1, ky0, ky1 = pl.pallas_call(
        _nms_body,
        out_shape=(shp, shp, shp, shp),
        scratch_shapes=[
            pltpu.VMEM((B, R, LANES), jnp.float32),
            pltpu.VMEM((B, R, LANES), jnp.float32),
            pltpu.VMEM((B, R, LANES), jnp.float32),
            pltpu.VMEM((B, R, LANES), jnp.float32),
            pltpu.VMEM((B, R, LANES), jnp.float32),
        ],
        interpret=interpret,
    )(scores, deltas, anch)
    out = jnp.stack([c.reshape(B, 8 * LANES)[:, :MAX_OUT]
                     for c in (kx0, kx1, ky0, ky1)], axis=-1)
    return out
